# baseline (speedup 1.0000x reference)
import dataclasses
import functools

import jax
import jax.numpy as jnp
from jax import lax
from jax.experimental import pallas as pl
from jax.experimental.pallas import tpu as pltpu
from jax.experimental.pallas import tpu_sc as plsc

N = 10000
E = 320000
D = 128

NC = 2
NS = 16
NW = NC * NS

CHUNK = 64
CPT = 160
PASSES = 4
CPP = CPT // PASSES
EPT = CPT * CHUNK
EPAD = NW * EPT
ACC_ROWS = N + 16
HIST = 10240
ROWS_A = 632
ROWS_LAST = N - 15 * ROWS_A
ZROWS_LAST = ACC_ROWS - 15 * ROWS_A

_mesh = plsc.VectorSubcoreMesh(core_axis_name="c", subcore_axis_name="s")

_sc_params = pltpu.CompilerParams()
if "needs_layout_passes" in pltpu.CompilerParams.__dataclass_fields__:
    _sc_params = dataclasses.replace(_sc_params, needs_layout_passes=False)


@functools.partial(
    pl.kernel,
    out_type=jax.ShapeDtypeStruct((NW * HIST,), jnp.float32),
    mesh=_mesh,
    scratch_types=[
        pltpu.VMEM((EPT,), jnp.int32),
        pltpu.VMEM((HIST,), jnp.float32),
    ],
    compiler_params=_sc_params,
)
def _degree_hist(dst_hbm, out_hbm, dstv, hist):
    c = lax.axis_index("c")
    s = lax.axis_index("s")
    wid = c * NS + s
    pltpu.sync_copy(dst_hbm.at[pl.ds(wid * EPT, EPT)], dstv)

    @pl.loop(0, HIST, step=16)
    def _(i):
        hist[pl.ds(i, 16)] = jnp.zeros((16,), jnp.float32)

    ones = jnp.ones((16,), jnp.float32)

    @pl.loop(0, EPT, step=16)
    def _(k):
        plsc.addupdate_scatter(hist, [dstv[pl.ds(k, 16)]], ones)

    pltpu.sync_copy(hist, out_hbm.at[pl.ds(wid * HIST, HIST)])


@functools.partial(
    pl.kernel,
    out_type=jax.ShapeDtypeStruct((NC, N, D), jnp.float32),
    mesh=_mesh,
    scratch_types=[
        pltpu.VMEM((CPP, CHUNK), jnp.int32),
        pltpu.VMEM((CPP, CHUNK), jnp.int32),
        pltpu.VMEM((CHUNK, D), jnp.float32),
        pltpu.VMEM((CHUNK, D), jnp.float32),
        pltpu.VMEM((CHUNK, D), jnp.float32),
        pltpu.VMEM((CHUNK, D), jnp.float32),
        pltpu.SemaphoreType.DMA,
        pltpu.SemaphoreType.DMA,
        pltpu.SemaphoreType.DMA,
        pltpu.SemaphoreType.DMA,
        pltpu.SemaphoreType.DMA,
        pltpu.SemaphoreType.DMA,
        pltpu.SemaphoreType.DMA,
        pltpu.SemaphoreType.DMA,
        pltpu.VMEM_SHARED((ACC_ROWS, D), jnp.float32),
    ],
)
def _propagate_rows(hp_hbm, src_hbm, dst_hbm, zeros_hbm, out_hbm,
                    srcv, dstv, buf0, buf1, buf2, buf3,
                    sg0, sg1, sg2, sg3, ss0, ss1, ss2, ss3, acc):
    c = lax.axis_index("c")
    s = lax.axis_index("s")
    wid = c * NS + s

    row0 = s * ROWS_A

    @pl.when(s < NS - 1)
    def _():
        pltpu.sync_copy(zeros_hbm.at[pl.ds(row0, ROWS_A)],
                        acc.at[pl.ds(row0, ROWS_A)])

    @pl.when(s == NS - 1)
    def _():
        pltpu.sync_copy(zeros_hbm.at[pl.ds(row0, ZROWS_LAST)],
                        acc.at[pl.ds(row0, ZROWS_LAST)])

    plsc.subcore_barrier()

    def _gather(j, buf, sem):
        pltpu.async_copy(hp_hbm.at[srcv.at[j]], buf, sem)

    def _wait_gather(j, buf, sem):
        pltpu.make_async_copy(hp_hbm.at[srcv.at[j]], buf, sem).wait()

    def _scatter(j, buf, sem):
        pltpu.async_copy(buf, acc.at[dstv.at[j]], sem, add=True)

    def _wait_scatter(j, buf, sem):
        pltpu.make_async_copy(buf, acc.at[dstv.at[j]], sem).wait()

    bufs = (buf0, buf1, buf2, buf3)
    sgs = (sg0, sg1, sg2, sg3)
    sss = (ss0, ss1, ss2, ss3)

    for p in range(PASSES):
        pltpu.sync_copy(src_hbm.at[pl.ds((wid * PASSES + p) * CPP, CPP)], srcv)
        pltpu.sync_copy(dst_hbm.at[pl.ds((wid * PASSES + p) * CPP, CPP)], dstv)

        _gather(0, buf0, sg0)
        _gather(1, buf1, sg1)

        @pl.loop(0, CPP, step=4)
        def _(j):
            for k in range(4):
                i = j + k
                nxt = (k + 2) % 4

                @pl.when(i + 2 < CPP)
                def _():
                    @pl.when(i >= 2)
                    def _():
                        _wait_scatter(i - 2, bufs[nxt], sss[nxt])

                    _gather(i + 2, bufs[nxt], sgs[nxt])

                _wait_gather(i, bufs[k], sgs[k])
                _scatter(i, bufs[k], sss[k])

        for k in range(4):
            _wait_scatter(CPP - 4 + k, bufs[k], sss[k])

    plsc.subcore_barrier()

    @pl.when(s < NS - 1)
    def _():
        pltpu.sync_copy(acc.at[pl.ds(row0, ROWS_A)],
                        out_hbm.at[c, pl.ds(row0, ROWS_A)])

    @pl.when(s == NS - 1)
    def _():
        pltpu.sync_copy(acc.at[pl.ds(row0, ROWS_LAST)],
                        out_hbm.at[c, pl.ds(row0, ROWS_LAST)])


@functools.partial(
    pl.kernel,
    out_type=jax.ShapeDtypeStruct((NW * HIST,), jnp.float32),
    mesh=_mesh,
    scratch_types=[
        pltpu.VMEM((EPT,), jnp.int32),
        pltpu.VMEM((EPT,), jnp.int32),
        pltpu.VMEM((N,), jnp.float32),
        pltpu.VMEM((HIST,), jnp.float32),
    ],
    compiler_params=_sc_params,
)
def _propagate_scalar(h2p_hbm, src_hbm, dst_hbm, out_hbm, srcv, dstv, table, hist):
    c = lax.axis_index("c")
    s = lax.axis_index("s")
    wid = c * NS + s
    pltpu.sync_copy(src_hbm.at[pl.ds(wid * EPT, EPT)], srcv)
    pltpu.sync_copy(dst_hbm.at[pl.ds(wid * EPT, EPT)], dstv)
    pltpu.sync_copy(h2p_hbm, table)

    @pl.loop(0, HIST, step=16)
    def _(i):
        hist[pl.ds(i, 16)] = jnp.zeros((16,), jnp.float32)

    @pl.loop(0, EPT, step=16)
    def _(k):
        vals = plsc.load_gather(table, [srcv[pl.ds(k, 16)]])
        plsc.addupdate_scatter(hist, [dstv[pl.ds(k, 16)]], vals)

    pltpu.sync_copy(hist, out_hbm.at[pl.ds(wid * HIST, HIST)])


def _tc_prep_body(x_ref, w1_ref, degc_ref, hp_ref, dinv_ref):
    deg = 1.0 + jnp.sum(degc_ref[...], axis=1, keepdims=True)
    dinv = lax.rsqrt(jnp.maximum(deg, 1e-12))
    h = jnp.dot(x_ref[...], w1_ref[...], preferred_element_type=jnp.float32)
    hp_ref[...] = h * dinv
    dinv_ref[...] = dinv


_tc_prep = pl.pallas_call(
    _tc_prep_body,
    out_shape=(jax.ShapeDtypeStruct((N, D), jnp.float32),
               jax.ShapeDtypeStruct((N, 1), jnp.float32)),
)


def _tc_mid_body(a0_ref, a1_ref, hp_ref, dinv_ref, b1_ref, w2_ref, h2p_ref):
    dinv = dinv_ref[...]
    z = dinv * (a0_ref[...] + a1_ref[...] + hp_ref[...]) + b1_ref[...]
    z = jnp.maximum(z, 0.0)
    h2 = jnp.dot(z, w2_ref[...], preferred_element_type=jnp.float32)
    h2p_ref[...] = h2 * dinv


_tc_mid = pl.pallas_call(
    _tc_mid_body,
    out_shape=jax.ShapeDtypeStruct((N, 1), jnp.float32),
)


def _tc_out_body(acc2c_ref, h2p_ref, dinv_ref, b2_ref, out_ref):
    ssum = jnp.sum(acc2c_ref[...], axis=1, keepdims=True)
    out_ref[...] = jax.nn.sigmoid(dinv_ref[...] * (ssum + h2p_ref[...])
                                  + b2_ref[...])


_tc_out = pl.pallas_call(
    _tc_out_body,
    out_shape=jax.ShapeDtypeStruct((N, 1), jnp.float32),
)


def kernel(x, edge_index, W1, b1, W2, b2):
    src = edge_index[0]
    dst = edge_index[1]

    pad_ids = lax.iota(jnp.int32, EPAD - E)
    src_p = jnp.concatenate([src, pad_ids % N])
    dst_p = jnp.concatenate([dst, N + (pad_ids % 16)])
    src2d = src_p.reshape(NW * CPT, CHUNK)
    dst2d = dst_p.reshape(NW * CPT, CHUNK)

    deg_flat = _degree_hist(dst_p)
    degc = deg_flat.reshape(NW, HIST)[:, :N].T

    hp, dinv = _tc_prep(x, W1, degc)
    zeros = jnp.zeros((ACC_ROWS, D), jnp.float32)
    accs = _propagate_rows(hp, src2d, dst2d, zeros)
    h2p = _tc_mid(accs[0], accs[1], hp, dinv, b1.reshape(1, D), W2)

    acc2_flat = _propagate_scalar(h2p.reshape(N), src_p, dst_p)
    acc2c = acc2_flat.reshape(NW, HIST)[:, :N].T

    return _tc_out(acc2c, h2p, dinv, b2.reshape(1, 1))


# speedup vs baseline: 1.0147x; 1.0147x over previous
import dataclasses
import functools

import jax
import jax.numpy as jnp
from jax import lax
from jax.experimental import pallas as pl
from jax.experimental.pallas import tpu as pltpu
from jax.experimental.pallas import tpu_sc as plsc

N = 10000
E = 320000
D = 128

NC = 2
NS = 16
NW = NC * NS

CHUNK = 128
CPT = 80
PASSES = 2
CPP = CPT // PASSES
EPT = CPT * CHUNK
EPAD = NW * EPT
ACC_ROWS = N + 16
HIST = 10240
ROWS_A = 632
ROWS_LAST = N - 15 * ROWS_A
ZROWS_LAST = ACC_ROWS - 15 * ROWS_A

_mesh = plsc.VectorSubcoreMesh(core_axis_name="c", subcore_axis_name="s")

_sc_params = pltpu.CompilerParams()
if "needs_layout_passes" in pltpu.CompilerParams.__dataclass_fields__:
    _sc_params = dataclasses.replace(_sc_params, needs_layout_passes=False)


@functools.partial(
    pl.kernel,
    out_type=jax.ShapeDtypeStruct((NW * HIST,), jnp.float32),
    mesh=_mesh,
    scratch_types=[
        pltpu.VMEM((EPT,), jnp.int32),
        pltpu.VMEM((HIST,), jnp.float32),
    ],
    compiler_params=_sc_params,
)
def _degree_hist(dst_hbm, out_hbm, dstv, hist):
    c = lax.axis_index("c")
    s = lax.axis_index("s")
    wid = c * NS + s
    pltpu.sync_copy(dst_hbm.at[pl.ds(wid * EPT, EPT)], dstv)

    @pl.loop(0, HIST, step=16)
    def _(i):
        hist[pl.ds(i, 16)] = jnp.zeros((16,), jnp.float32)

    ones = jnp.ones((16,), jnp.float32)

    @pl.loop(0, EPT, step=16)
    def _(k):
        plsc.addupdate_scatter(hist, [dstv[pl.ds(k, 16)]], ones)

    pltpu.sync_copy(hist, out_hbm.at[pl.ds(wid * HIST, HIST)])


@functools.partial(
    pl.kernel,
    out_type=jax.ShapeDtypeStruct((NC, N, D), jnp.float32),
    mesh=_mesh,
    scratch_types=[
        pltpu.VMEM((CPP, CHUNK), jnp.int32),
        pltpu.VMEM((CPP, CHUNK), jnp.int32),
        pltpu.VMEM((CHUNK, D), jnp.float32),
        pltpu.VMEM((CHUNK, D), jnp.float32),
        pltpu.SemaphoreType.DMA,
        pltpu.SemaphoreType.DMA,
        pltpu.SemaphoreType.DMA,
        pltpu.SemaphoreType.DMA,
        pltpu.VMEM_SHARED((ACC_ROWS, D), jnp.float32),
    ],
)
def _propagate_rows(hp_hbm, src_hbm, dst_hbm, zeros_hbm, out_hbm,
                    srcv, dstv, buf0, buf1, sg0, sg1, ss0, ss1, acc):
    c = lax.axis_index("c")
    s = lax.axis_index("s")
    wid = c * NS + s

    row0 = s * ROWS_A

    @pl.when(s < NS - 1)
    def _():
        pltpu.sync_copy(zeros_hbm.at[pl.ds(row0, ROWS_A)],
                        acc.at[pl.ds(row0, ROWS_A)])

    @pl.when(s == NS - 1)
    def _():
        pltpu.sync_copy(zeros_hbm.at[pl.ds(row0, ZROWS_LAST)],
                        acc.at[pl.ds(row0, ZROWS_LAST)])

    plsc.subcore_barrier()

    def _gather(j, buf, sem):
        pltpu.async_copy(hp_hbm.at[srcv.at[j]], buf, sem)

    def _wait_gather(j, buf, sem):
        pltpu.make_async_copy(hp_hbm.at[srcv.at[j]], buf, sem).wait()

    def _scatter(j, buf, sem):
        pltpu.async_copy(buf, acc.at[dstv.at[j]], sem, add=True)

    def _wait_scatter(j, buf, sem):
        pltpu.make_async_copy(buf, acc.at[dstv.at[j]], sem).wait()

    for p in range(PASSES):
        pltpu.sync_copy(src_hbm.at[pl.ds((wid * PASSES + p) * CPP, CPP)], srcv)
        pltpu.sync_copy(dst_hbm.at[pl.ds((wid * PASSES + p) * CPP, CPP)], dstv)

        _gather(0, buf0, sg0)

        @pl.loop(0, CPP, step=2)
        def _(j):
            @pl.when(j > 0)
            def _():
                _wait_scatter(j - 1, buf1, ss1)

            _gather(j + 1, buf1, sg1)
            _wait_gather(j, buf0, sg0)
            _scatter(j, buf0, ss0)
            _wait_scatter(j, buf0, ss0)

            @pl.when(j + 2 < CPP)
            def _():
                _gather(j + 2, buf0, sg0)

            _wait_gather(j + 1, buf1, sg1)
            _scatter(j + 1, buf1, ss1)

        _wait_scatter(CPP - 1, buf1, ss1)

    plsc.subcore_barrier()

    @pl.when(s < NS - 1)
    def _():
        pltpu.sync_copy(acc.at[pl.ds(row0, ROWS_A)],
                        out_hbm.at[c, pl.ds(row0, ROWS_A)])

    @pl.when(s == NS - 1)
    def _():
        pltpu.sync_copy(acc.at[pl.ds(row0, ROWS_LAST)],
                        out_hbm.at[c, pl.ds(row0, ROWS_LAST)])


SLICE = HIST // NS
EPT2 = EPAD // NS


@functools.partial(
    pl.kernel,
    out_type=jax.ShapeDtypeStruct((HIST,), jnp.float32),
    mesh=_mesh,
    scratch_types=[
        pltpu.VMEM((EPT2,), jnp.int32),
        pltpu.VMEM((EPT2,), jnp.int32),
        pltpu.VMEM((N,), jnp.float32),
        pltpu.VMEM((HIST,), jnp.float32),
        pltpu.VMEM((NS, SLICE), jnp.float32),
        pltpu.VMEM((SLICE,), jnp.float32),
        pltpu.VMEM((SLICE,), jnp.float32),
        pltpu.VMEM((SLICE,), jnp.float32),
        pltpu.VMEM_SHARED((NS, HIST), jnp.float32),
    ],
    compiler_params=_sc_params,
)
def _propagate_scalar_out(h2p_hbm, dinv_hbm, w_hbm, src_hbm, dst_hbm, out_hbm,
                          srcv, dstv, table, hist, red, dinvv, wv, outv,
                          staging):
    c = lax.axis_index("c")
    s = lax.axis_index("s")

    @pl.when(c == 0)
    def _():
        pltpu.sync_copy(src_hbm.at[pl.ds(s * EPT2, EPT2)], srcv)
        pltpu.sync_copy(dst_hbm.at[pl.ds(s * EPT2, EPT2)], dstv)
        pltpu.sync_copy(h2p_hbm, table)

        @pl.loop(0, HIST, step=16)
        def _(i):
            hist[pl.ds(i, 16)] = jnp.zeros((16,), jnp.float32)

        @pl.loop(0, EPT2, step=16)
        def _(k):
            vals = plsc.load_gather(table, [srcv[pl.ds(k, 16)]])
            plsc.addupdate_scatter(hist, [dstv[pl.ds(k, 16)]], vals)

        pltpu.sync_copy(hist, staging.at[s])
        plsc.subcore_barrier()

        pltpu.sync_copy(staging.at[:, pl.ds(s * SLICE, SLICE)], red)
        pltpu.sync_copy(dinv_hbm.at[pl.ds(s * SLICE, SLICE)], dinvv)
        pltpu.sync_copy(w_hbm.at[pl.ds(s * SLICE, SLICE)], wv)

        @pl.loop(0, SLICE, step=16)
        def _(g):
            a = red[0, pl.ds(g, 16)]
            for r in range(1, NS):
                a = a + red[r, pl.ds(g, 16)]
            xx = dinvv[pl.ds(g, 16)] * a + wv[pl.ds(g, 16)]
            outv[pl.ds(g, 16)] = 1.0 / (1.0 + jnp.exp(-xx))

        pltpu.sync_copy(outv, out_hbm.at[pl.ds(s * SLICE, SLICE)])


def _tc_mm_body(x_ref, w1_ref, h1_ref):
    h1_ref[...] = jnp.dot(x_ref[...], w1_ref[...],
                          preferred_element_type=jnp.float32)


_tc_mm = pl.pallas_call(
    _tc_mm_body,
    out_shape=jax.ShapeDtypeStruct((N, D), jnp.float32),
)


def _tc_scale_body(h1_ref, degc_ref, hp_ref, dinv_ref):
    deg = 1.0 + jnp.sum(degc_ref[...], axis=1, keepdims=True)
    dinv = lax.rsqrt(jnp.maximum(deg, 1e-12))
    hp_ref[...] = h1_ref[...] * dinv
    dinv_ref[...] = dinv


_tc_scale = pl.pallas_call(
    _tc_scale_body,
    out_shape=(jax.ShapeDtypeStruct((N, D), jnp.float32),
               jax.ShapeDtypeStruct((N, 1), jnp.float32)),
)


def _tc_mid_body(a0_ref, a1_ref, hp_ref, dinv_ref, b1_ref, w2_ref, b2_ref,
                 h2p_ref, w_ref):
    dinv = dinv_ref[...]
    z = dinv * (a0_ref[...] + a1_ref[...] + hp_ref[...]) + b1_ref[...]
    z = jnp.maximum(z, 0.0)
    h2 = jnp.dot(z, w2_ref[...], preferred_element_type=jnp.float32)
    h2p = h2 * dinv
    h2p_ref[...] = h2p
    w_ref[...] = h2p * dinv + b2_ref[...]


_tc_mid = pl.pallas_call(
    _tc_mid_body,
    out_shape=(jax.ShapeDtypeStruct((N, 1), jnp.float32),
               jax.ShapeDtypeStruct((N, 1), jnp.float32)),
)


def kernel(x, edge_index, W1, b1, W2, b2):
    src = edge_index[0]
    dst = edge_index[1]

    pad_ids = lax.iota(jnp.int32, EPAD - E)
    src_p = jnp.concatenate([src, pad_ids % N])
    dst_p = jnp.concatenate([dst, N + (pad_ids % 16)])
    src2d = src_p.reshape(NW * CPT, CHUNK)
    dst2d = dst_p.reshape(NW * CPT, CHUNK)

    deg_flat = _degree_hist(dst_p)
    h1 = _tc_mm(x, W1)
    degc = deg_flat.reshape(NW, HIST)[:, :N].T

    hp, dinv = _tc_scale(h1, degc)
    zeros = jnp.zeros((ACC_ROWS, D), jnp.float32)
    accs = _propagate_rows(hp, src2d, dst2d, zeros)
    h2p, w = _tc_mid(accs[0], accs[1], hp, dinv, b1.reshape(1, D), W2,
                     b2.reshape(1, 1))

    dinv_pad = jnp.pad(dinv.reshape(N), (0, HIST - N))
    w_pad = jnp.pad(w.reshape(N), (0, HIST - N))
    out1d = _propagate_scalar_out(h2p.reshape(N), dinv_pad, w_pad,
                                  src_p, dst_p)
    return out1d[:N].reshape(N, 1)


# speedup vs baseline: 1.0607x; 1.0454x over previous
import dataclasses
import functools

import jax
import jax.numpy as jnp
import numpy as np
from jax import lax
from jax.experimental import pallas as pl
from jax.experimental.pallas import tpu as pltpu
from jax.experimental.pallas import tpu_sc as plsc

N = 10000
E = 320000
D = 128

NC = 2
NS = 16
NW = NC * NS

CHUNK = 128
CPT = 80
PASSES = 2
CPP = CPT // PASSES
EPT = CPT * CHUNK
EPAD = NW * EPT
ACC_ROWS = N + 16
HIST = 10240
ROWS_A = 632
ROWS_LAST = N - 15 * ROWS_A
ZROWS_LAST = ACC_ROWS - 15 * ROWS_A

ER = E // CHUNK
TAIL_ROW0 = 2400
TAIL_ROWS = NW * CPT - TAIL_ROW0

_PAD_IDS = np.arange((NW * CPT - ER) * CHUNK, dtype=np.int32)
_PAD_SRC = _PAD_IDS % N
_PAD_DST = N + (_PAD_IDS % 16)

_mesh = plsc.VectorSubcoreMesh(core_axis_name="c", subcore_axis_name="s")

_sc_params = pltpu.CompilerParams()
if "needs_layout_passes" in pltpu.CompilerParams.__dataclass_fields__:
    _sc_params = dataclasses.replace(_sc_params, needs_layout_passes=False)


@functools.partial(
    pl.kernel,
    out_type=jax.ShapeDtypeStruct((NC * HIST,), jnp.float32),
    mesh=_mesh,
    scratch_types=[
        pltpu.VMEM((CPT, CHUNK), jnp.int32),
        pltpu.VMEM((HIST,), jnp.float32),
        pltpu.VMEM((NS, HIST // NS), jnp.float32),
        pltpu.VMEM((HIST // NS,), jnp.float32),
        pltpu.VMEM_SHARED((NS, HIST), jnp.float32),
    ],
    compiler_params=_sc_params,
)
def _degree_hist(edges_hbm, tail_dst_hbm, out_hbm, dstv, hist, red, combv,
                 staging):
    c = lax.axis_index("c")
    s = lax.axis_index("s")
    wid = c * NS + s
    base = wid * CPT

    @pl.when(base < TAIL_ROW0)
    def _():
        pltpu.sync_copy(edges_hbm.at[1, pl.ds(base, CPT)], dstv)

    @pl.when(base >= TAIL_ROW0)
    def _():
        pltpu.sync_copy(tail_dst_hbm.at[pl.ds(base - TAIL_ROW0, CPT)], dstv)

    @pl.loop(0, HIST, step=16)
    def _(i):
        hist[pl.ds(i, 16)] = jnp.zeros((16,), jnp.float32)

    ones = jnp.ones((16,), jnp.float32)

    @pl.loop(0, CPT)
    def _(r):
        @pl.loop(0, CHUNK, step=16)
        def _(k):
            plsc.addupdate_scatter(hist, [dstv[r, pl.ds(k, 16)]], ones)

    pltpu.sync_copy(hist, staging.at[s])
    plsc.subcore_barrier()
    pltpu.sync_copy(staging.at[:, pl.ds(s * SLICE, SLICE)], red)

    @pl.loop(0, SLICE, step=16)
    def _(g):
        a = red[0, pl.ds(g, 16)]
        for r in range(1, NS):
            a = a + red[r, pl.ds(g, 16)]
        combv[pl.ds(g, 16)] = a

    pltpu.sync_copy(combv, out_hbm.at[pl.ds(c * HIST + s * SLICE, SLICE)])


@functools.partial(
    pl.kernel,
    out_type=jax.ShapeDtypeStruct((NC, N, D), jnp.float32),
    mesh=_mesh,
    scratch_types=[
        pltpu.VMEM((CPP, CHUNK), jnp.int32),
        pltpu.VMEM((CPP, CHUNK), jnp.int32),
        pltpu.VMEM((CHUNK, D), jnp.float32),
        pltpu.VMEM((CHUNK, D), jnp.float32),
        pltpu.SemaphoreType.DMA,
        pltpu.SemaphoreType.DMA,
        pltpu.SemaphoreType.DMA,
        pltpu.SemaphoreType.DMA,
        pltpu.VMEM_SHARED((ACC_ROWS, D), jnp.float32),
    ],
)
def _propagate_rows(hp_hbm, edges_hbm, tail_src_hbm, tail_dst_hbm, zeros_hbm,
                    out_hbm, srcv, dstv, buf0, buf1, sg0, sg1, ss0, ss1, acc):
    c = lax.axis_index("c")
    s = lax.axis_index("s")
    wid = c * NS + s

    row0 = s * ROWS_A

    @pl.when(s < NS - 1)
    def _():
        pltpu.sync_copy(zeros_hbm, acc.at[pl.ds(row0, ROWS_A)])

    @pl.when(s == NS - 1)
    def _():
        pltpu.sync_copy(zeros_hbm.at[pl.ds(0, ZROWS_LAST)],
                        acc.at[pl.ds(row0, ZROWS_LAST)])

    plsc.subcore_barrier()

    def _gather(j, buf, sem):
        pltpu.async_copy(hp_hbm.at[srcv.at[j]], buf, sem)

    def _wait_gather(j, buf, sem):
        pltpu.make_async_copy(hp_hbm.at[srcv.at[j]], buf, sem).wait()

    def _scatter(j, buf, sem):
        pltpu.async_copy(buf, acc.at[dstv.at[j]], sem, add=True)

    def _wait_scatter(j, buf, sem):
        pltpu.make_async_copy(buf, acc.at[dstv.at[j]], sem).wait()

    for p in range(PASSES):
        base = wid * CPT + p * CPP

        @pl.when(base < TAIL_ROW0)
        def _():
            pltpu.sync_copy(edges_hbm.at[0, pl.ds(base, CPP)], srcv)
            pltpu.sync_copy(edges_hbm.at[1, pl.ds(base, CPP)], dstv)

        @pl.when(base >= TAIL_ROW0)
        def _():
            pltpu.sync_copy(tail_src_hbm.at[pl.ds(base - TAIL_ROW0, CPP)], srcv)
            pltpu.sync_copy(tail_dst_hbm.at[pl.ds(base - TAIL_ROW0, CPP)], dstv)

        _gather(0, buf0, sg0)

        @pl.loop(0, CPP, step=2)
        def _(j):
            @pl.when(j > 0)
            def _():
                _wait_scatter(j - 1, buf1, ss1)

            _gather(j + 1, buf1, sg1)
            _wait_gather(j, buf0, sg0)
            _scatter(j, buf0, ss0)
            _wait_scatter(j, buf0, ss0)

            @pl.when(j + 2 < CPP)
            def _():
                _gather(j + 2, buf0, sg0)

            _wait_gather(j + 1, buf1, sg1)
            _scatter(j + 1, buf1, ss1)

        _wait_scatter(CPP - 1, buf1, ss1)

    plsc.subcore_barrier()

    @pl.when(s < NS - 1)
    def _():
        pltpu.sync_copy(acc.at[pl.ds(row0, ROWS_A)],
                        out_hbm.at[c, pl.ds(row0, ROWS_A)])

    @pl.when(s == NS - 1)
    def _():
        pltpu.sync_copy(acc.at[pl.ds(row0, ROWS_LAST)],
                        out_hbm.at[c, pl.ds(row0, ROWS_LAST)])


SLICE = HIST // NS
CPT2 = NW * CPT // NS


@functools.partial(
    pl.kernel,
    out_type=jax.ShapeDtypeStruct((HIST,), jnp.float32),
    mesh=_mesh,
    scratch_types=[
        pltpu.VMEM((CPT2, CHUNK), jnp.int32),
        pltpu.VMEM((CPT2, CHUNK), jnp.int32),
        pltpu.VMEM((N,), jnp.float32),
        pltpu.VMEM((HIST,), jnp.float32),
        pltpu.VMEM((NS, SLICE), jnp.float32),
        pltpu.VMEM((SLICE,), jnp.float32),
        pltpu.VMEM((SLICE,), jnp.float32),
        pltpu.VMEM((SLICE,), jnp.float32),
        pltpu.VMEM_SHARED((NS, HIST), jnp.float32),
    ],
    compiler_params=_sc_params,
)
def _propagate_scalar_out(h2p_hbm, dinv_hbm, w_hbm, edges_hbm, tail_src_hbm,
                          tail_dst_hbm, out_hbm,
                          srcv, dstv, table, hist, red, dinvv, wv, outv,
                          staging):
    c = lax.axis_index("c")
    s = lax.axis_index("s")

    @pl.when(c == 0)
    def _():
        base = s * CPT2

        @pl.when(base < TAIL_ROW0)
        def _():
            pltpu.sync_copy(edges_hbm.at[0, pl.ds(base, CPT2)], srcv)
            pltpu.sync_copy(edges_hbm.at[1, pl.ds(base, CPT2)], dstv)

        @pl.when(base >= TAIL_ROW0)
        def _():
            pltpu.sync_copy(tail_src_hbm, srcv)
            pltpu.sync_copy(tail_dst_hbm, dstv)

        pltpu.sync_copy(h2p_hbm, table)

        @pl.loop(0, HIST, step=16)
        def _(i):
            hist[pl.ds(i, 16)] = jnp.zeros((16,), jnp.float32)

        @pl.loop(0, CPT2)
        def _(r):
            @pl.loop(0, CHUNK, step=16)
            def _(k):
                vals = plsc.load_gather(table, [srcv[r, pl.ds(k, 16)]])
                plsc.addupdate_scatter(hist, [dstv[r, pl.ds(k, 16)]], vals)

        pltpu.sync_copy(hist, staging.at[s])
        plsc.subcore_barrier()

        pltpu.sync_copy(staging.at[:, pl.ds(s * SLICE, SLICE)], red)
        pltpu.sync_copy(dinv_hbm.at[pl.ds(s * SLICE, SLICE)], dinvv)
        pltpu.sync_copy(w_hbm.at[pl.ds(s * SLICE, SLICE)], wv)

        @pl.loop(0, SLICE, step=16)
        def _(g):
            a = red[0, pl.ds(g, 16)]
            for r in range(1, NS):
                a = a + red[r, pl.ds(g, 16)]
            xx = dinvv[pl.ds(g, 16)] * a + wv[pl.ds(g, 16)]
            outv[pl.ds(g, 16)] = 1.0 / (1.0 + jnp.exp(-xx))

        pltpu.sync_copy(outv, out_hbm.at[pl.ds(s * SLICE, SLICE)])


def _tc_mm_body(x_ref, w1_ref, h1_ref):
    h1_ref[...] = jnp.dot(x_ref[...], w1_ref[...],
                          preferred_element_type=jnp.float32)


_tc_mm = pl.pallas_call(
    _tc_mm_body,
    out_shape=jax.ShapeDtypeStruct((N, D), jnp.float32),
)


def _tc_scale_body(h1_ref, degc_ref, hp_ref, dinv_ref):
    deg = 1.0 + jnp.sum(degc_ref[...], axis=1, keepdims=True)
    dinv = lax.rsqrt(jnp.maximum(deg, 1e-12))
    hp_ref[...] = h1_ref[...] * dinv
    dinv_ref[...] = dinv


_tc_scale = pl.pallas_call(
    _tc_scale_body,
    out_shape=(jax.ShapeDtypeStruct((N, D), jnp.float32),
               jax.ShapeDtypeStruct((N, 1), jnp.float32)),
)


def _tc_mid_body(a0_ref, a1_ref, hp_ref, dinv_ref, b1_ref, w2_ref, b2_ref,
                 h2p_ref, w_ref):
    dinv = dinv_ref[...]
    z = dinv * (a0_ref[...] + a1_ref[...] + hp_ref[...]) + b1_ref[...]
    z = jnp.maximum(z, 0.0)
    h2 = jnp.dot(z, w2_ref[...], preferred_element_type=jnp.float32)
    h2p = h2 * dinv
    h2p_ref[...] = h2p
    w_ref[...] = h2p * dinv + b2_ref[...]


_tc_mid = pl.pallas_call(
    _tc_mid_body,
    out_shape=(jax.ShapeDtypeStruct((N, 1), jnp.float32),
               jax.ShapeDtypeStruct((N, 1), jnp.float32)),
)


def kernel(x, edge_index, W1, b1, W2, b2):
    edges3d = edge_index.reshape(2, ER, CHUNK)
    tail_src = jnp.concatenate(
        [edge_index[0, TAIL_ROW0 * CHUNK:], jnp.asarray(_PAD_SRC)]
    ).reshape(TAIL_ROWS, CHUNK)
    tail_dst = jnp.concatenate(
        [edge_index[1, TAIL_ROW0 * CHUNK:], jnp.asarray(_PAD_DST)]
    ).reshape(TAIL_ROWS, CHUNK)

    deg_flat = _degree_hist(edges3d, tail_dst)
    h1 = _tc_mm(x, W1)
    degc = deg_flat.reshape(NC, HIST)[:, :N].T

    hp, dinv = _tc_scale(h1, degc)
    zeros = jnp.zeros((ROWS_A, D), jnp.float32)
    accs = _propagate_rows(hp, edges3d, tail_src, tail_dst, zeros)
    h2p, w = _tc_mid(accs[0], accs[1], hp, dinv, b1.reshape(1, D), W2,
                     b2.reshape(1, 1))

    dinv_pad = jnp.pad(dinv.reshape(N), (0, HIST - N))
    w_pad = jnp.pad(w.reshape(N), (0, HIST - N))
    out1d = _propagate_scalar_out(h2p.reshape(N), dinv_pad, w_pad,
                                  edges3d, tail_src, tail_dst)
    return out1d[:N].reshape(N, 1)


# speedup vs baseline: 1.0640x; 1.0031x over previous
import dataclasses
import functools

import jax
import jax.numpy as jnp
import numpy as np
from jax import lax
from jax.experimental import pallas as pl
from jax.experimental.pallas import tpu as pltpu
from jax.experimental.pallas import tpu_sc as plsc

N = 10000
E = 320000
D = 128

NC = 2
NS = 16
NW = NC * NS

CHUNK = 128
CPT = 80
PASSES = 2
CPP = CPT // PASSES
EPT = CPT * CHUNK
EPAD = NW * EPT
ACC_ROWS = N + 16
HIST = 10240
ROWS_A = 632
ROWS_LAST = N - 15 * ROWS_A
ZROWS_LAST = ACC_ROWS - 15 * ROWS_A

ER = E // CHUNK
TAIL_ROW0 = 2400
TAIL_ROWS = NW * CPT - TAIL_ROW0

_PAD_IDS = np.arange((NW * CPT - ER) * CHUNK, dtype=np.int32)
_PAD_SRC = _PAD_IDS % N
_PAD_DST = N + (_PAD_IDS % 16)

_mesh = plsc.VectorSubcoreMesh(core_axis_name="c", subcore_axis_name="s")

_sc_params = pltpu.CompilerParams()
if "needs_layout_passes" in pltpu.CompilerParams.__dataclass_fields__:
    _sc_params = dataclasses.replace(_sc_params, needs_layout_passes=False)


@functools.partial(
    pl.kernel,
    out_type=jax.ShapeDtypeStruct((NW * HIST,), jnp.float32),
    mesh=_mesh,
    scratch_types=[
        pltpu.VMEM((CPT, CHUNK), jnp.int32),
        pltpu.VMEM((HIST,), jnp.float32),
    ],
    compiler_params=_sc_params,
)
def _degree_hist(edges_hbm, tail_dst_hbm, out_hbm, dstv, hist):
    c = lax.axis_index("c")
    s = lax.axis_index("s")
    wid = c * NS + s
    base = wid * CPT

    @pl.when(base < TAIL_ROW0)
    def _():
        pltpu.sync_copy(edges_hbm.at[1, pl.ds(base, CPT)], dstv)

    @pl.when(base >= TAIL_ROW0)
    def _():
        pltpu.sync_copy(tail_dst_hbm.at[pl.ds(base - TAIL_ROW0, CPT)], dstv)

    @pl.loop(0, HIST, step=16)
    def _(i):
        hist[pl.ds(i, 16)] = jnp.zeros((16,), jnp.float32)

    ones = jnp.ones((16,), jnp.float32)

    @pl.loop(0, CPT)
    def _(r):
        @pl.loop(0, CHUNK, step=16)
        def _(k):
            plsc.addupdate_scatter(hist, [dstv[r, pl.ds(k, 16)]], ones)

    pltpu.sync_copy(hist, out_hbm.at[pl.ds(wid * HIST, HIST)])


@functools.partial(
    pl.kernel,
    out_type=jax.ShapeDtypeStruct((NC, N, D), jnp.float32),
    mesh=_mesh,
    scratch_types=[
        pltpu.VMEM((CPP, CHUNK), jnp.int32),
        pltpu.VMEM((CPP, CHUNK), jnp.int32),
        pltpu.VMEM((CHUNK, D), jnp.float32),
        pltpu.VMEM((CHUNK, D), jnp.float32),
        pltpu.SemaphoreType.DMA,
        pltpu.SemaphoreType.DMA,
        pltpu.SemaphoreType.DMA,
        pltpu.SemaphoreType.DMA,
        pltpu.VMEM_SHARED((ACC_ROWS, D), jnp.float32),
    ],
)
def _propagate_rows(hp_hbm, edges_hbm, tail_src_hbm, tail_dst_hbm, zeros_hbm,
                    out_hbm, srcv, dstv, buf0, buf1, sg0, sg1, ss0, ss1, acc):
    c = lax.axis_index("c")
    s = lax.axis_index("s")
    wid = c * NS + s

    row0 = s * ROWS_A

    @pl.when(s < NS - 1)
    def _():
        pltpu.sync_copy(zeros_hbm, acc.at[pl.ds(row0, ROWS_A)])

    @pl.when(s == NS - 1)
    def _():
        pltpu.sync_copy(zeros_hbm.at[pl.ds(0, ZROWS_LAST)],
                        acc.at[pl.ds(row0, ZROWS_LAST)])

    plsc.subcore_barrier()

    def _gather(j, buf, sem):
        pltpu.async_copy(hp_hbm.at[srcv.at[j]], buf, sem)

    def _wait_gather(j, buf, sem):
        pltpu.make_async_copy(hp_hbm.at[srcv.at[j]], buf, sem).wait()

    def _scatter(j, buf, sem):
        pltpu.async_copy(buf, acc.at[dstv.at[j]], sem, add=True)

    def _wait_scatter(j, buf, sem):
        pltpu.make_async_copy(buf, acc.at[dstv.at[j]], sem).wait()

    for p in range(PASSES):
        base = wid * CPT + p * CPP

        @pl.when(base < TAIL_ROW0)
        def _():
            pltpu.sync_copy(edges_hbm.at[0, pl.ds(base, CPP)], srcv)
            pltpu.sync_copy(edges_hbm.at[1, pl.ds(base, CPP)], dstv)

        @pl.when(base >= TAIL_ROW0)
        def _():
            pltpu.sync_copy(tail_src_hbm.at[pl.ds(base - TAIL_ROW0, CPP)], srcv)
            pltpu.sync_copy(tail_dst_hbm.at[pl.ds(base - TAIL_ROW0, CPP)], dstv)

        _gather(0, buf0, sg0)

        @pl.loop(0, CPP, step=2)
        def _(j):
            @pl.when(j > 0)
            def _():
                _wait_scatter(j - 1, buf1, ss1)

            _gather(j + 1, buf1, sg1)
            _wait_gather(j, buf0, sg0)
            _scatter(j, buf0, ss0)
            _wait_scatter(j, buf0, ss0)

            @pl.when(j + 2 < CPP)
            def _():
                _gather(j + 2, buf0, sg0)

            _wait_gather(j + 1, buf1, sg1)
            _scatter(j + 1, buf1, ss1)

        _wait_scatter(CPP - 1, buf1, ss1)

    plsc.subcore_barrier()

    @pl.when(s < NS - 1)
    def _():
        pltpu.sync_copy(acc.at[pl.ds(row0, ROWS_A)],
                        out_hbm.at[c, pl.ds(row0, ROWS_A)])

    @pl.when(s == NS - 1)
    def _():
        pltpu.sync_copy(acc.at[pl.ds(row0, ROWS_LAST)],
                        out_hbm.at[c, pl.ds(row0, ROWS_LAST)])


SLICE = HIST // NS
CPT2 = NW * CPT // NS


@functools.partial(
    pl.kernel,
    out_type=jax.ShapeDtypeStruct((HIST,), jnp.float32),
    mesh=_mesh,
    scratch_types=[
        pltpu.VMEM((CPT2, CHUNK), jnp.int32),
        pltpu.VMEM((CPT2, CHUNK), jnp.int32),
        pltpu.VMEM((N,), jnp.float32),
        pltpu.VMEM((HIST,), jnp.float32),
        pltpu.VMEM((NS, SLICE), jnp.float32),
        pltpu.VMEM((SLICE,), jnp.float32),
        pltpu.VMEM((SLICE,), jnp.float32),
        pltpu.VMEM((SLICE,), jnp.float32),
        pltpu.VMEM_SHARED((NS, HIST), jnp.float32),
    ],
    compiler_params=_sc_params,
)
def _propagate_scalar_out(h2p_hbm, dinv_hbm, w_hbm, edges_hbm, tail_src_hbm,
                          tail_dst_hbm, out_hbm,
                          srcv, dstv, table, hist, red, dinvv, wv, outv,
                          staging):
    c = lax.axis_index("c")
    s = lax.axis_index("s")

    @pl.when(c == 0)
    def _():
        base = s * CPT2

        @pl.when(base < TAIL_ROW0)
        def _():
            pltpu.sync_copy(edges_hbm.at[0, pl.ds(base, CPT2)], srcv)
            pltpu.sync_copy(edges_hbm.at[1, pl.ds(base, CPT2)], dstv)

        @pl.when(base >= TAIL_ROW0)
        def _():
            pltpu.sync_copy(tail_src_hbm, srcv)
            pltpu.sync_copy(tail_dst_hbm, dstv)

        pltpu.sync_copy(h2p_hbm, table)

        @pl.loop(0, HIST, step=16)
        def _(i):
            hist[pl.ds(i, 16)] = jnp.zeros((16,), jnp.float32)

        @pl.loop(0, CPT2)
        def _(r):
            @pl.loop(0, CHUNK, step=16)
            def _(k):
                vals = plsc.load_gather(table, [srcv[r, pl.ds(k, 16)]])
                plsc.addupdate_scatter(hist, [dstv[r, pl.ds(k, 16)]], vals)

        pltpu.sync_copy(hist, staging.at[s])
        plsc.subcore_barrier()

        pltpu.sync_copy(staging.at[:, pl.ds(s * SLICE, SLICE)], red)
        pltpu.sync_copy(dinv_hbm.at[pl.ds(s * SLICE, SLICE)], dinvv)
        pltpu.sync_copy(w_hbm.at[pl.ds(s * SLICE, SLICE)], wv)

        @pl.loop(0, SLICE, step=16)
        def _(g):
            a = red[0, pl.ds(g, 16)]
            for r in range(1, NS):
                a = a + red[r, pl.ds(g, 16)]
            xx = dinvv[pl.ds(g, 16)] * a + wv[pl.ds(g, 16)]
            outv[pl.ds(g, 16)] = 1.0 / (1.0 + jnp.exp(-xx))

        pltpu.sync_copy(outv, out_hbm.at[pl.ds(s * SLICE, SLICE)])


def _tc_mm_body(x_ref, w1_ref, h1_ref):
    h1_ref[...] = jnp.dot(x_ref[...], w1_ref[...],
                          preferred_element_type=jnp.float32)


_tc_mm = pl.pallas_call(
    _tc_mm_body,
    out_shape=jax.ShapeDtypeStruct((N, D), jnp.float32),
)


def _tc_scale_body(h1_ref, degc_ref, hp_ref, dinv_ref):
    deg = 1.0 + jnp.sum(degc_ref[...], axis=1, keepdims=True)
    dinv = lax.rsqrt(jnp.maximum(deg, 1e-12))
    hp_ref[...] = h1_ref[...] * dinv
    dinv_ref[...] = dinv


_tc_scale = pl.pallas_call(
    _tc_scale_body,
    out_shape=(jax.ShapeDtypeStruct((N, D), jnp.float32),
               jax.ShapeDtypeStruct((N, 1), jnp.float32)),
)


def _tc_mid_body(a0_ref, a1_ref, hp_ref, dinv_ref, b1_ref, w2_ref, b2_ref,
                 h2p_ref, w_ref):
    dinv = dinv_ref[...]
    z = dinv * (a0_ref[...] + a1_ref[...] + hp_ref[...]) + b1_ref[...]
    z = jnp.maximum(z, 0.0)
    h2 = jnp.dot(z, w2_ref[...], preferred_element_type=jnp.float32)
    h2p = h2 * dinv
    h2p_ref[...] = h2p
    w_ref[...] = h2p * dinv + b2_ref[...]


_tc_mid = pl.pallas_call(
    _tc_mid_body,
    out_shape=(jax.ShapeDtypeStruct((N, 1), jnp.float32),
               jax.ShapeDtypeStruct((N, 1), jnp.float32)),
)


def kernel(x, edge_index, W1, b1, W2, b2):
    edges3d = edge_index.reshape(2, ER, CHUNK)
    tail_src = jnp.concatenate(
        [edge_index[0, TAIL_ROW0 * CHUNK:], jnp.asarray(_PAD_SRC)]
    ).reshape(TAIL_ROWS, CHUNK)
    tail_dst = jnp.concatenate(
        [edge_index[1, TAIL_ROW0 * CHUNK:], jnp.asarray(_PAD_DST)]
    ).reshape(TAIL_ROWS, CHUNK)

    deg_flat = _degree_hist(edges3d, tail_dst)
    h1 = _tc_mm(x, W1)
    degc = deg_flat.reshape(NW, HIST)[:, :N].T

    hp, dinv = _tc_scale(h1, degc)
    zeros = jnp.zeros((ROWS_A, D), jnp.float32)
    accs = _propagate_rows(hp, edges3d, tail_src, tail_dst, zeros)
    h2p, w = _tc_mid(accs[0], accs[1], hp, dinv, b1.reshape(1, D), W2,
                     b2.reshape(1, 1))

    dinv_pad = jnp.pad(dinv.reshape(N), (0, HIST - N))
    w_pad = jnp.pad(w.reshape(N), (0, HIST - N))
    out1d = _propagate_scalar_out(h2p.reshape(N), dinv_pad, w_pad,
                                  edges3d, tail_src, tail_dst)
    return out1d[:N].reshape(N, 1)
